# Initial kernel scaffold; baseline (speedup 1.0000x reference)
import jax
import jax.numpy as jnp
from jax.experimental import pallas as pl


def kernel(x, W):
    raise NotImplementedError("write your pallas kernel here")


# speedup vs baseline: 20.8594x; 20.8594x over previous
import functools

import jax
import jax.numpy as jnp
from jax import lax
from jax.experimental import pallas as pl
from jax.experimental.pallas import tpu as pltpu

NUM_EXPERTS = 64
TOP_K = 8
HIDDEN = 768
TOKENS = 32768

BLOCK_T = 1024


def _router_block(x_ref, w_ref, gates_ref, map_ref):
    x = x_ref[...]
    w = w_ref[...]
    logits = lax.dot_general(
        x, w, (((1,), (1,)), ((), ())),
        preferred_element_type=jnp.float32)

    row_max = jnp.max(logits, axis=-1, keepdims=True)
    ex = jnp.exp(logits - row_max)
    probs = ex / jnp.sum(ex, axis=-1, keepdims=True)

    t = jnp.max(probs, axis=-1, keepdims=True)
    for _ in range(TOP_K - 1):
        masked = jnp.where(probs < t, probs, -1.0)
        t = jnp.maximum(jnp.max(masked, axis=-1, keepdims=True), 0.0)

    gt = probs > t
    cnt_gt = jnp.sum(gt.astype(jnp.float32), axis=-1, keepdims=True)
    eq = probs == t
    col = lax.broadcasted_iota(jnp.int32, (NUM_EXPERTS, NUM_EXPERTS), 1)
    row = lax.broadcasted_iota(jnp.int32, (NUM_EXPERTS, NUM_EXPERTS), 0)
    tri = (row <= col).astype(jnp.float32)
    rank = jnp.dot(eq.astype(jnp.float32), tri,
                   preferred_element_type=jnp.float32)
    sel = gt | (eq & ((rank + cnt_gt) <= TOP_K))

    gates_ref[...] = jnp.where(sel, probs, 0.0)
    map_ref[...] = sel.astype(jnp.int32)


@jax.jit
def kernel(x, W):
    n_blocks = TOKENS // BLOCK_T
    gates, topk_map = pl.pallas_call(
        _router_block,
        grid=(n_blocks,),
        in_specs=[
            pl.BlockSpec((BLOCK_T, HIDDEN), lambda i: (i, 0)),
            pl.BlockSpec((NUM_EXPERTS, HIDDEN), lambda i: (0, 0)),
        ],
        out_specs=[
            pl.BlockSpec((BLOCK_T, NUM_EXPERTS), lambda i: (i, 0)),
            pl.BlockSpec((BLOCK_T, NUM_EXPERTS), lambda i: (i, 0)),
        ],
        out_shape=[
            jax.ShapeDtypeStruct((TOKENS, NUM_EXPERTS), jnp.float32),
            jax.ShapeDtypeStruct((TOKENS, NUM_EXPERTS), jnp.int32),
        ],
    )(x, W)
    return (gates, topk_map)
